# baseline (speedup 1.0000x reference)
import jax
import jax.numpy as jnp
from jax.experimental import pallas as pl
from jax.experimental.pallas import tpu as pltpu

NI = 128
H = 2 * NI
NF = 256
NUM_ROUNDS = 2
BM = 1024


def _body(hf_ref, wih_ref, whh_ref, bih_ref, bhh_ref,
          fcW_ref, fcb_ref, l2W_ref, l2b_ref, out_ref,
          Mb_ref, gbias_ref, fcWb_ref):
    @pl.when(pl.program_id(0) == 0)
    def _prep():
        wih = wih_ref[...]
        whh = whh_ref[...]
        wsw = jnp.concatenate([wih[:, NI:H], wih[:, 0:NI]], axis=1)
        Mb_ref[...] = jnp.concatenate(
            [wsw[0:2 * H] + whh[0:2 * H], wsw[2 * H:3 * H],
             whh[2 * H:3 * H]], axis=0).astype(jnp.bfloat16).T
        bih = bih_ref[...]
        bhh = bhh_ref[...]
        gbias_ref[...] = jnp.concatenate(
            [bih[:, 0:2 * H] + bhh[:, 0:2 * H], bih[:, 2 * H:3 * H],
             bhh[:, 2 * H:3 * H]], axis=1)
        fcWb_ref[...] = fcW_ref[...].astype(jnp.bfloat16).T

    f = hf_ref[...]
    h0 = f[:, 0, :].astype(jnp.bfloat16)
    h1 = f[:, 1, :].astype(jnp.bfloat16)
    Mb = Mb_ref[...]
    gbias = gbias_ref[...].astype(jnp.bfloat16)
    half = jnp.bfloat16(0.5)
    one = jnp.bfloat16(1.0)
    dn = (((1,), (0,)), ((), ()))
    for _ in range(NUM_ROUNDS):
        g = (jax.lax.dot_general(h0, Mb[0:NI], dn,
                                 preferred_element_type=jnp.float32)
             + jax.lax.dot_general(h1, Mb[NI:H], dn,
                                   preferred_element_type=jnp.float32)
             ).astype(jnp.bfloat16) + gbias
        rz = half * (one + jnp.tanh(half * g[:, 0:2 * H]))
        n = jnp.tanh(g[:, 2 * H:3 * H] + rz[:, 0:H] * g[:, 3 * H:4 * H])
        n0 = n[:, 0:NI]
        n1 = n[:, NI:H]
        h0 = n0 + rz[:, H:H + NI] * (h0 - n0)
        h1 = n1 + rz[:, H + NI:2 * H] * (h1 - n1)
    fcWb = fcWb_ref[...]
    y = (jax.lax.dot_general(h0, fcWb[0:NI], dn,
                             preferred_element_type=jnp.float32)
         + jax.lax.dot_general(h1, fcWb[NI:H], dn,
                               preferred_element_type=jnp.float32)
         ) + fcb_ref[...]
    y = jnp.maximum(y, 0.0)
    out_ref[...] = (jnp.sum(y * l2W_ref[...], axis=1, keepdims=True)
                    + l2b_ref[...])


@jax.jit
def kernel(features, W_ih, W_hh, b_ih, b_hh, fc_W, fc_b, l2_W, l2_b):
    bs = features.shape[0]

    grid = (bs // BM,)
    const = lambda i: (0, 0)
    out = pl.pallas_call(
        _body,
        grid=grid,
        in_specs=[
            pl.BlockSpec((BM, 2, NI), lambda i: (i, 0, 0)),
            pl.BlockSpec((3 * H, H), const),
            pl.BlockSpec((3 * H, H), const),
            pl.BlockSpec((1, 3 * H), const),
            pl.BlockSpec((1, 3 * H), const),
            pl.BlockSpec((NF, H), const),
            pl.BlockSpec((1, NF), const),
            pl.BlockSpec((1, NF), const),
            pl.BlockSpec((1, 1), const),
        ],
        out_specs=pl.BlockSpec((BM, 1), lambda i: (i, 0)),
        out_shape=jax.ShapeDtypeStruct((bs, 1), jnp.float32),
        scratch_shapes=[
            pltpu.VMEM((H, 4 * H), jnp.bfloat16),
            pltpu.VMEM((1, 4 * H), jnp.float32),
            pltpu.VMEM((H, NF), jnp.bfloat16),
        ],
        compiler_params=pltpu.CompilerParams(
            dimension_semantics=("arbitrary",),
        ),
    )(features, W_ih, W_hh, b_ih.reshape(1, 3 * H), b_hh.reshape(1, 3 * H),
      fc_W, fc_b.reshape(1, NF), l2_W, l2_b.reshape(1, 1))
    return out


# speedup vs baseline: 1.5336x; 1.5336x over previous
import jax
import jax.numpy as jnp
from jax.experimental import pallas as pl
from jax.experimental.pallas import tpu as pltpu

NI = 128
H = 2 * NI
NF = 256
NUM_ROUNDS = 2
BM = 1024


def _body(hf_ref, wih_ref, whh_ref, bih_ref, bhh_ref,
          fcW_ref, fcb_ref, l2W_ref, l2b_ref, out_ref,
          Mb_ref, gbias_ref, fcWb_ref):
    @pl.when(pl.program_id(0) == 0)
    def _prep():
        wih = wih_ref[...]
        whh = whh_ref[...]
        wsw = jnp.concatenate([wih[:, NI:H], wih[:, 0:NI]], axis=1)
        Mb_ref[...] = jnp.concatenate(
            [wsw[0:2 * H] + whh[0:2 * H], wsw[2 * H:3 * H],
             whh[2 * H:3 * H]], axis=0).astype(jnp.bfloat16).T
        bih = bih_ref[...]
        bhh = bhh_ref[...]
        gbias_ref[...] = jnp.concatenate(
            [bih[:, 0:2 * H] + bhh[:, 0:2 * H], bih[:, 2 * H:3 * H],
             bhh[:, 2 * H:3 * H]], axis=1)
        fcWb_ref[...] = fcW_ref[...].astype(jnp.bfloat16).T

    f = hf_ref[...]
    h = jnp.concatenate([f[:, 0, :], f[:, 1, :]],
                        axis=1).astype(jnp.bfloat16)
    Mb = Mb_ref[...]
    gbias = gbias_ref[...].astype(jnp.bfloat16)
    half = jnp.bfloat16(0.5)
    one = jnp.bfloat16(1.0)
    dn = (((1,), (0,)), ((), ()))
    for _ in range(NUM_ROUNDS):
        g = jax.lax.dot_general(h, Mb, dn,
                                preferred_element_type=jnp.float32
                                ).astype(jnp.bfloat16) + gbias
        rz = half * (one + jnp.tanh(half * g[:, 0:2 * H]))
        n = jnp.tanh(g[:, 2 * H:3 * H] + rz[:, 0:H] * g[:, 3 * H:4 * H])
        h = n + rz[:, H:2 * H] * (h - n)
    y = jax.lax.dot_general(h, fcWb_ref[...], dn,
                            preferred_element_type=jnp.float32) + fcb_ref[...]
    y = jnp.maximum(y, 0.0)
    out_ref[...] = (jnp.sum(y * l2W_ref[...], axis=1, keepdims=True)
                    + l2b_ref[...])


@jax.jit
def kernel(features, W_ih, W_hh, b_ih, b_hh, fc_W, fc_b, l2_W, l2_b):
    bs = features.shape[0]

    grid = (bs // BM,)
    const = lambda i: (0, 0)
    out = pl.pallas_call(
        _body,
        grid=grid,
        in_specs=[
            pl.BlockSpec((BM, 2, NI), lambda i: (i, 0, 0)),
            pl.BlockSpec((3 * H, H), const),
            pl.BlockSpec((3 * H, H), const),
            pl.BlockSpec((1, 3 * H), const),
            pl.BlockSpec((1, 3 * H), const),
            pl.BlockSpec((NF, H), const),
            pl.BlockSpec((1, NF), const),
            pl.BlockSpec((1, NF), const),
            pl.BlockSpec((1, 1), const),
        ],
        out_specs=pl.BlockSpec((BM, 1), lambda i: (i, 0)),
        out_shape=jax.ShapeDtypeStruct((bs, 1), jnp.float32),
        scratch_shapes=[
            pltpu.VMEM((H, 4 * H), jnp.bfloat16),
            pltpu.VMEM((1, 4 * H), jnp.float32),
            pltpu.VMEM((H, NF), jnp.bfloat16),
        ],
        compiler_params=pltpu.CompilerParams(
            dimension_semantics=("arbitrary",),
        ),
    )(features, W_ih, W_hh, b_ih.reshape(1, 3 * H), b_hh.reshape(1, 3 * H),
      fc_W, fc_b.reshape(1, NF), l2_W, l2_b.reshape(1, 1))
    return out
